# Initial kernel scaffold; baseline (speedup 1.0000x reference)
import jax
import jax.numpy as jnp
from jax.experimental import pallas as pl


def kernel(x, token_table, pos_table):
    raise NotImplementedError("write your pallas kernel here")


# speedup vs baseline: 1.3896x; 1.3896x over previous
import functools

import jax
import jax.numpy as jnp
from jax import lax
from jax.experimental import pallas as pl
from jax.experimental.pallas import tpu as pltpu
from jax.experimental.pallas import tpu_sc as plsc

L = 200
D = 32
NC, NS = 2, 16
NW = NC * NS

SEQ_PER_CHUNK = 4
CH_ROWS = SEQ_PER_CHUNK * L
IDX_G = 100
N_G = CH_ROWS // IDX_G


def _emb_call(x2d, token_table, pos_table, total_rows):
    rows_per_w = total_rows // NW
    n_chunks = rows_per_w // CH_ROWS
    mesh = plsc.VectorSubcoreMesh(core_axis_name="c", subcore_axis_name="s")

    @functools.partial(
        pl.kernel,
        out_type=jax.ShapeDtypeStruct((total_rows, D), jnp.float32),
        mesh=mesh,
        compiler_params=pltpu.CompilerParams(use_tc_tiling_on_sc=False),
        scratch_types=[
            pltpu.VMEM((N_G, IDX_G), jnp.int32),
            pltpu.VMEM((CH_ROWS, D), jnp.float32),
            pltpu.VMEM((L, D), jnp.float32),
            pltpu.SemaphoreType.DMA,
        ],
    )
    def body(x_hbm, tok_hbm, pos_hbm, out_hbm, idx_v, rows_v, pos_v, sem):
        wid = lax.axis_index("s") * NC + lax.axis_index("c")
        base = wid * rows_per_w
        pltpu.sync_copy(pos_hbm, pos_v)

        def chunk(ci, carry):
            row0 = base + ci * CH_ROWS
            idx_row0 = wid * (rows_per_w // IDX_G) + ci * N_G
            pltpu.sync_copy(x_hbm.at[pl.ds(idx_row0, N_G)], idx_v)
            descs = [
                pltpu.async_copy(
                    tok_hbm.at[idx_v.at[g]],
                    rows_v.at[pl.ds(g * IDX_G, IDX_G)],
                    sem,
                )
                for g in range(N_G)
            ]
            for d in descs:
                d.wait()

            def row(r, c2):
                for s in range(SEQ_PER_CHUNK):
                    rr = s * L + r
                    for h in (0, 16):
                        rows_v[rr, pl.ds(h, 16)] = (
                            rows_v[rr, pl.ds(h, 16)] + pos_v[r, pl.ds(h, 16)]
                        )
                return c2

            lax.fori_loop(0, L, row, 0)
            pltpu.sync_copy(rows_v, out_hbm.at[pl.ds(row0, CH_ROWS)])
            return carry

        lax.fori_loop(0, n_chunks, chunk, 0)

    return body(x2d, token_table, pos_table)


def kernel(x, token_table, pos_table):
    batch, maxlen = x.shape
    total_rows = batch * maxlen
    x2d = x.reshape(total_rows // IDX_G, IDX_G)
    out = _emb_call(x2d, token_table, pos_table, total_rows)
    return out.reshape(batch, maxlen, D)


# speedup vs baseline: 1.4885x; 1.0712x over previous
import functools

import jax
import jax.numpy as jnp
from jax import lax
from jax.experimental import pallas as pl
from jax.experimental.pallas import tpu as pltpu
from jax.experimental.pallas import tpu_sc as plsc

L = 200
D = 32
NC, NS = 2, 16
NW = NC * NS

SEQ_PER_CHUNK = 4
CH_ROWS = SEQ_PER_CHUNK * L
IDX_G = 100
N_G = CH_ROWS // IDX_G


def _emb_call(x2d, token_table, pos_table, total_rows):
    rows_per_w = total_rows // NW
    n_chunks = rows_per_w // CH_ROWS
    idx_rows_w = rows_per_w // IDX_G
    mesh = plsc.VectorSubcoreMesh(core_axis_name="c", subcore_axis_name="s")

    @functools.partial(
        pl.kernel,
        out_type=jax.ShapeDtypeStruct((total_rows, D), jnp.float32),
        mesh=mesh,
        compiler_params=pltpu.CompilerParams(use_tc_tiling_on_sc=False),
        scratch_types=[
            pltpu.VMEM((idx_rows_w, IDX_G), jnp.int32),
            pltpu.VMEM((CH_ROWS, D), jnp.float32),
            pltpu.VMEM((CH_ROWS, D), jnp.float32),
            pltpu.VMEM((L, D), jnp.float32),
            pltpu.SemaphoreType.DMA,
            pltpu.SemaphoreType.DMA,
            pltpu.SemaphoreType.DMA,
            pltpu.SemaphoreType.DMA,
        ],
    )
    def body(x_hbm, tok_hbm, pos_hbm, out_hbm, idx_v, rows0, rows1, pos_v,
             sg0, sg1, st0, st1):
        rows_v = (rows0, rows1)
        sem_g = (sg0, sg1)
        sem_st = (st0, st1)
        wid = lax.axis_index("s") * NC + lax.axis_index("c")
        base = wid * rows_per_w
        pltpu.sync_copy(x_hbm.at[pl.ds(wid * idx_rows_w, idx_rows_w)], idx_v)
        pltpu.sync_copy(pos_hbm, pos_v)

        def fire_gathers(ci, s):
            for g in range(N_G):
                pltpu.async_copy(
                    tok_hbm.at[idx_v.at[ci * N_G + g]],
                    rows_v[s].at[pl.ds(g * IDX_G, IDX_G)],
                    sem_g[s],
                )

        def drain_gathers(ci, s):
            for g in range(N_G):
                pltpu.make_async_copy(
                    tok_hbm.at[idx_v.at[ci * N_G + g]],
                    rows_v[s].at[pl.ds(g * IDX_G, IDX_G)],
                    sem_g[s],
                ).wait()

        def add_pos(s):
            def row(r, c2):
                for q in range(SEQ_PER_CHUNK):
                    rr = q * L + r
                    for h in (0, 16):
                        rows_v[s][rr, pl.ds(h, 16)] = (
                            rows_v[s][rr, pl.ds(h, 16)] + pos_v[r, pl.ds(h, 16)]
                        )
                return c2

            lax.fori_loop(0, L, row, 0)

        def store(ci, s):
            pltpu.async_copy(
                rows_v[s],
                out_hbm.at[pl.ds(base + ci * CH_ROWS, CH_ROWS)],
                sem_st[s],
            )

        def wait_store(ci, s):
            pltpu.make_async_copy(
                rows_v[s],
                out_hbm.at[pl.ds(base + ci * CH_ROWS, CH_ROWS)],
                sem_st[s],
            ).wait()

        fire_gathers(0, 0)

        @pl.loop(0, n_chunks, step=2)
        def chunks(ci):
            for b in range(2):
                c = ci + b
                s, t = b, 1 - b

                @pl.when(c + 1 < n_chunks)
                def _prefetch():
                    @pl.when(c >= 1)
                    def _reuse_guard():
                        wait_store(c - 1, t)

                    fire_gathers(c + 1, t)

                drain_gathers(c, s)
                add_pos(s)
                store(c, s)

        wait_store(n_chunks - 2, 0)
        wait_store(n_chunks - 1, 1)

    return body(x2d, token_table, pos_table)


def kernel(x, token_table, pos_table):
    batch, maxlen = x.shape
    total_rows = batch * maxlen
    x2d = x.reshape(total_rows // IDX_G, IDX_G)
    out = _emb_call(x2d, token_table, pos_table, total_rows)
    return out.reshape(batch, maxlen, D)
